# Initial kernel scaffold; baseline (speedup 1.0000x reference)
import jax
import jax.numpy as jnp
from jax.experimental import pallas as pl


def kernel(x, edge_index, edge_weight, W1, b1, W2, b2, W3, b3, W4, b4, Wl, bl):
    raise NotImplementedError("write your pallas kernel here")


# speedup vs baseline: 3.6390x; 3.6390x over previous
import functools

import jax
import jax.numpy as jnp
from jax import lax
from jax.experimental import pallas as pl
from jax.experimental.pallas import tpu as pltpu
from jax.experimental.pallas import tpu_sc as plsc

N = 10000
F = 128
NCLASS = 40
E = 320000

NC = 2
NS = 16
NW = NC * NS

CHUNK = 128
CHUNKS_PER_W = 79
EPW = CHUNK * CHUNKS_PER_W
PE = EPW * NW

ROWS_PER_TILE = 632
NPAD = ROWS_PER_TILE * NS


_sc_mesh = plsc.VectorSubcoreMesh(core_axis_name="c", subcore_axis_name="s")


@functools.partial(
    pl.kernel,
    out_type=jax.ShapeDtypeStruct((NC * NPAD, F), jnp.float32),
    mesh=_sc_mesh,
    scratch_types=[
        pltpu.VMEM((CHUNK,), jnp.int32),
        pltpu.VMEM((CHUNK,), jnp.int32),
        pltpu.VMEM((CHUNK, F), jnp.float32),
        pltpu.VMEM_SHARED((NPAD, F), jnp.float32),
        pltpu.SemaphoreType.DMA,
    ],
)
def _sc_aggregate(hw_hbm, src_hbm, dst_hbm, zeros_hbm, out_hbm,
                  src_v, dst_v, rows_v, acc_sh, sem):
    c = lax.axis_index("c")
    s = lax.axis_index("s")
    pltpu.sync_copy(zeros_hbm.at[pl.ds(s * ROWS_PER_TILE, ROWS_PER_TILE)],
                    acc_sh.at[pl.ds(s * ROWS_PER_TILE, ROWS_PER_TILE)])
    plsc.subcore_barrier()

    wid = s * NC + c
    ebase = wid * EPW

    def body(j, carry):
        base = ebase + j * CHUNK
        pltpu.sync_copy(src_hbm.at[pl.ds(base, CHUNK)], src_v)
        pltpu.sync_copy(dst_hbm.at[pl.ds(base, CHUNK)], dst_v)
        pltpu.async_copy(hw_hbm.at[src_v], rows_v, sem).wait()
        pltpu.sync_copy(rows_v, acc_sh.at[dst_v], add=True)
        return carry

    lax.fori_loop(0, CHUNKS_PER_W, body, 0)
    plsc.subcore_barrier()
    pltpu.sync_copy(acc_sh.at[pl.ds(s * ROWS_PER_TILE, ROWS_PER_TILE)],
                    out_hbm.at[pl.ds(c * NPAD + s * ROWS_PER_TILE, ROWS_PER_TILE)])


_GRID = 4
_BLK = NPAD // _GRID


def _mm_body(x_ref, w_ref, o_ref):
    o_ref[...] = jnp.dot(x_ref[...], w_ref[...],
                         preferred_element_type=jnp.float32,
                         precision=lax.Precision.HIGHEST)


def _tc_matmul(x, w):
    return pl.pallas_call(
        _mm_body,
        grid=(_GRID,),
        in_specs=[pl.BlockSpec((_BLK, F), lambda i: (i, 0)),
                  pl.BlockSpec((F, F), lambda i: (0, 0))],
        out_specs=pl.BlockSpec((_BLK, F), lambda i: (i, 0)),
        out_shape=jax.ShapeDtypeStruct((NPAD, F), jnp.float32),
    )(x, w)


def _post_agg(a0, a1, b):
    acc = a0 + a1 + b
    n = jnp.sqrt(jnp.sum(acc * acc, axis=-1, keepdims=True))
    return jnp.maximum(acc / jnp.maximum(n, 1e-12), 0.0)


def _layer_body(a0_ref, a1_ref, b_ref, w_ref, o_ref):
    t = _post_agg(a0_ref[...], a1_ref[...], b_ref[...])
    o_ref[...] = jnp.dot(t, w_ref[...], preferred_element_type=jnp.float32,
                         precision=lax.Precision.HIGHEST)


def _tc_layer(agg2, b, w):
    return pl.pallas_call(
        _layer_body,
        grid=(_GRID,),
        in_specs=[pl.BlockSpec((_BLK, F), lambda i: (i, 0)),
                  pl.BlockSpec((_BLK, F), lambda i: (i + _GRID, 0)),
                  pl.BlockSpec((1, F), lambda i: (0, 0)),
                  pl.BlockSpec((F, F), lambda i: (0, 0))],
        out_specs=pl.BlockSpec((_BLK, F), lambda i: (i, 0)),
        out_shape=jax.ShapeDtypeStruct((NPAD, F), jnp.float32),
    )(agg2, agg2, b, w)


def _final_body(a0_ref, a1_ref, b_ref, wl_ref, bl_ref,
                x4_ref, logits_ref, probs_ref):
    x4 = _post_agg(a0_ref[...], a1_ref[...], b_ref[...])
    x4_ref[...] = x4
    logits = jnp.dot(x4, wl_ref[...], preferred_element_type=jnp.float32,
                     precision=lax.Precision.HIGHEST) + bl_ref[...]
    logits_ref[...] = logits
    m = jnp.max(logits, axis=-1, keepdims=True)
    e = jnp.exp(logits - m)
    probs_ref[...] = e / jnp.sum(e, axis=-1, keepdims=True)


def _tc_final(agg2, b, wl, bl):
    return pl.pallas_call(
        _final_body,
        grid=(_GRID,),
        in_specs=[pl.BlockSpec((_BLK, F), lambda i: (i, 0)),
                  pl.BlockSpec((_BLK, F), lambda i: (i + _GRID, 0)),
                  pl.BlockSpec((1, F), lambda i: (0, 0)),
                  pl.BlockSpec((F, NCLASS), lambda i: (0, 0)),
                  pl.BlockSpec((1, NCLASS), lambda i: (0, 0))],
        out_specs=[pl.BlockSpec((_BLK, F), lambda i: (i, 0)),
                   pl.BlockSpec((_BLK, NCLASS), lambda i: (i, 0)),
                   pl.BlockSpec((_BLK, NCLASS), lambda i: (i, 0))],
        out_shape=[jax.ShapeDtypeStruct((NPAD, F), jnp.float32),
                   jax.ShapeDtypeStruct((NPAD, NCLASS), jnp.float32),
                   jax.ShapeDtypeStruct((NPAD, NCLASS), jnp.float32)],
    )(agg2, agg2, b, wl, bl)


def kernel(x, edge_index, edge_weight, W1, b1, W2, b2, W3, b3, W4, b4, Wl, bl):
    del edge_weight
    src = edge_index[0]
    dst = edge_index[1]
    src_p = jnp.concatenate([src, jnp.zeros((PE - E,), jnp.int32)])
    dst_p = jnp.concatenate([dst, jnp.full((PE - E,), N, jnp.int32)])
    zeros = jnp.zeros((NPAD, F), jnp.float32)
    xp = jnp.concatenate([x, jnp.zeros((NPAD - N, F), jnp.float32)])

    b1r, b2r, b3r, b4r = (b.reshape(1, F) for b in (b1, b2, b3, b4))
    blr = bl.reshape(1, NCLASS)

    hw = _tc_matmul(xp, W1)
    agg = _sc_aggregate(hw, src_p, dst_p, zeros)
    hw = _tc_layer(agg, b1r, W2)
    agg = _sc_aggregate(hw, src_p, dst_p, zeros)
    hw = _tc_layer(agg, b2r, W3)
    agg = _sc_aggregate(hw, src_p, dst_p, zeros)
    hw = _tc_layer(agg, b3r, W4)
    agg = _sc_aggregate(hw, src_p, dst_p, zeros)
    x4, logits, probs = _tc_final(agg, b4r, Wl, blr)

    return (logits[:N], probs[:N], x4[:N])
